# baseline (speedup 1.0000x reference)
import jax
import jax.numpy as jnp
from jax import lax
from jax.experimental import pallas as pl
from jax.experimental.pallas import tpu as pltpu
from jax.experimental.pallas import tpu_sc as plsc

N = 10000
E = 320000
EPS = 1e-8

NW = 32
EPW = E // NW
CH = 80
NIT = EPW // CH
TW = 256
VW = 128

NBLK = 1000
EBLK = 2000


def _nn(sumsq):
    return jnp.sqrt(jnp.clip(sumsq, EPS, None))


def _bd(w, reps):
    k, n = w.shape
    m = jnp.zeros((reps * k, reps * n), w.dtype)
    for c in range(reps):
        m = m.at[c * k:(c + 1) * k, c * n:(c + 1) * n].set(w)
    return m


def _bd_perm(w):
    m = jnp.zeros((6, 6), w.dtype)
    for c in range(3):
        for i in range(2):
            for j in range(2):
                m = m.at[2 * c + i, 3 * j + c].set(w[i, j])
    return m


def _full(shape):
    return pl.BlockSpec(shape, lambda i: tuple(0 for _ in shape))


def _node_pre_body(x_ref, xv_ref, lsw_ref, lsb_ref, wa_ref, wc_ref, b1_ref,
                   wh1_ref, s_ref, tsrc_ref, tdst_ref):
    s = jnp.dot(x_ref[...], lsw_ref[...], preferred_element_type=jnp.float32)
    s = s + lsb_ref[...]
    s_ref[...] = s
    sa = jnp.dot(s, wa_ref[...], preferred_element_type=jnp.float32) + b1_ref[...]
    sc = jnp.dot(s, wc_ref[...], preferred_element_type=jnp.float32)
    xv = xv_ref[...]
    wh1 = wh1_ref[...]
    zpad = jnp.zeros((xv.shape[0], TW - 143), jnp.float32)
    pcols, qcols = [], []
    for c in range(3):
        vc = jnp.concatenate([xv[:, c:c + 1], xv[:, 3 + c:4 + c]], axis=1)
        pcols.append(jnp.dot(vc, wh1[0:2, :], preferred_element_type=jnp.float32))
        qcols.append(jnp.dot(vc, wh1[3:5, :], preferred_element_type=jnp.float32))
    tsrc_ref[...] = jnp.concatenate([sa] + pcols + [zpad], axis=1)
    tdst_ref[...] = jnp.concatenate([sc] + qcols + [zpad], axis=1)


def _node_pre(x, xv6, lsw, lsb, wa, wc, b1, wh1, interpret=False):
    grid = N // NBLK
    return pl.pallas_call(
        _node_pre_body,
        grid=(grid,),
        in_specs=[
            pl.BlockSpec((NBLK, 1433), lambda i: (i, 0)),
            pl.BlockSpec((NBLK, 6), lambda i: (i, 0)),
            _full((1433, 128)),
            _full((1, 128)),
            _full((128, 128)),
            _full((128, 128)),
            _full((1, 128)),
            _full((5, 5)),
        ],
        out_specs=[
            pl.BlockSpec((NBLK, 128), lambda i: (i, 0)),
            pl.BlockSpec((NBLK, TW), lambda i: (i, 0)),
            pl.BlockSpec((NBLK, TW), lambda i: (i, 0)),
        ],
        out_shape=[
            jax.ShapeDtypeStruct((N, 128), jnp.float32),
            jax.ShapeDtypeStruct((N, TW), jnp.float32),
            jax.ShapeDtypeStruct((N, TW), jnp.float32),
        ],
        interpret=interpret,
    )(x, xv6, lsw, lsb, wa, wc, b1, wh1)


def _sc_gather_body(tsrc_hbm, tdst_hbm, srci_hbm, dsti_hbm, gs_hbm, gd_hbm,
                    idx_v, rows_v, sem):
    c = lax.axis_index("c")
    s = lax.axis_index("s")
    wid = s * 2 + c
    base = wid * EPW

    def one_table(tab_hbm, ih_hbm, out_hbm):
        pltpu.sync_copy(ih_hbm.at[wid], idx_v)

        def body(j, carry):
            pltpu.async_copy(tab_hbm.at[idx_v.at[j]], rows_v, sem).wait()
            pltpu.sync_copy(rows_v, out_hbm.at[pl.ds(base + j * CH, CH)])
            return carry

        lax.fori_loop(0, NIT, body, 0)

    one_table(tsrc_hbm, srci_hbm, gs_hbm)
    one_table(tdst_hbm, dsti_hbm, gd_hbm)


def _sc_gather(tsrc, tdst, srci, dsti):
    mesh = plsc.VectorSubcoreMesh(core_axis_name="c", subcore_axis_name="s")
    fn = pl.kernel(
        _sc_gather_body,
        out_type=[
            jax.ShapeDtypeStruct((E, TW), jnp.float32),
            jax.ShapeDtypeStruct((E, TW), jnp.float32),
        ],
        mesh=mesh,
        scratch_types=[
            pltpu.VMEM((NIT, CH), jnp.int32),
            pltpu.VMEM((CH, TW), jnp.float32),
            pltpu.SemaphoreType.DMA,
        ],
    )
    return fn(tsrc, tdst, srci, dsti)


def _edge_body(gs_ref, gsv_ref, gd_ref, gdv_ref, ep_ref, wh1r2_ref, wb_ref,
               wd_ref, wv1_ref, wh2_ref, w2a_ref, w2b_ref, b2_ref, wv2_ref,
               wh3_ref, w3a_ref, w3b_ref, b3_ref, wv3_ref, outs_ref, outv_ref):
    gsv = gsv_ref[...]
    gdv = gdv_ref[...]
    ep = ep_ref[...]
    es = ep[:, 0:8]
    nb = es.shape[0]
    bf = jnp.bfloat16
    evt = jnp.concatenate([ep[:, 8 + c:9 + c] * wh1r2_ref[...]
                           for c in range(3)], axis=1)
    vh1 = gsv[:, 0:15] + gdv[:, 0:15] + evt
    vn1 = _nn(vh1[:, 0:5] ** 2 + vh1[:, 5:10] ** 2 + vh1[:, 10:15] ** 2)
    s1 = gs_ref[...] + gd_ref[...]
    s1 = s1 + jnp.dot(es, wb_ref[...], preferred_element_type=jnp.float32)
    s1 = s1 + jnp.dot(vn1, wd_ref[...], preferred_element_type=jnp.float32)
    s1 = jnp.maximum(s1, 0.0)
    vo1 = jnp.dot(vh1, wv1_ref[...], preferred_element_type=jnp.float32)
    g1 = jax.nn.sigmoid(_nn(vo1[:, 0:2] ** 2 + vo1[:, 2:4] ** 2
                            + vo1[:, 4:6] ** 2))
    vo1 = vo1 * jnp.concatenate([g1, g1, g1], axis=1)
    vh2 = jnp.dot(vo1, wh2_ref[...], preferred_element_type=jnp.float32)
    vn2 = _nn(vh2[:, 0:2] ** 2 + vh2[:, 2:4] ** 2 + vh2[:, 4:6] ** 2)
    s2 = jnp.dot(s1.astype(bf), w2a_ref[...],
                 preferred_element_type=jnp.float32)
    s2 = s2 + jnp.dot(vn2, w2b_ref[...], preferred_element_type=jnp.float32)
    s2 = jnp.maximum(s2 + b2_ref[...], 0.0)
    vo2 = jnp.dot(vh2, wv2_ref[...], preferred_element_type=jnp.float32)
    g2 = jax.nn.sigmoid(_nn(vo2[:, 0:2] ** 2 + vo2[:, 2:4] ** 2
                            + vo2[:, 4:6] ** 2))
    vo2 = vo2 * jnp.concatenate([g2, g2, g2], axis=1)
    vh3 = jnp.dot(vo2, wh3_ref[...], preferred_element_type=jnp.float32)
    vn3 = _nn(vh3[:, 0:2] ** 2 + vh3[:, 2:4] ** 2 + vh3[:, 4:6] ** 2)
    s3 = jnp.dot(s2.astype(bf), w3a_ref[...],
                 preferred_element_type=jnp.float32)
    s3 = s3 + jnp.dot(vn3, w3b_ref[...], preferred_element_type=jnp.float32)
    outs_ref[...] = s3 + b3_ref[...]
    vo3 = jnp.dot(vh3, wv3_ref[...], preferred_element_type=jnp.float32)
    ones = jnp.ones((nb, 1), jnp.float32)
    zpad = jnp.zeros((nb, VW - 7), jnp.float32)
    outv_ref[...] = jnp.concatenate([vo3, ones, zpad], axis=1)


def _edge(gs, gd, epack, w, interpret=False):
    grid = E // EBLK
    return pl.pallas_call(
        _edge_body,
        grid=(grid,),
        in_specs=[
            pl.BlockSpec((EBLK, 128), lambda i: (i, 0)),
            pl.BlockSpec((EBLK, 128), lambda i: (i, 1)),
            pl.BlockSpec((EBLK, 128), lambda i: (i, 0)),
            pl.BlockSpec((EBLK, 128), lambda i: (i, 1)),
            pl.BlockSpec((EBLK, 16), lambda i: (i, 0)),
            _full((1, 5)),
            _full((8, 128)),
            _full((5, 128)),
            _full((15, 6)),
            _full((6, 6)),
            _full((128, 128)),
            _full((2, 128)),
            _full((1, 128)),
            _full((6, 6)),
            _full((6, 6)),
            _full((128, 128)),
            _full((2, 128)),
            _full((1, 128)),
            _full((6, 6)),
        ],
        out_specs=[
            pl.BlockSpec((EBLK, 128), lambda i: (i, 0)),
            pl.BlockSpec((EBLK, VW), lambda i: (i, 0)),
        ],
        out_shape=[
            jax.ShapeDtypeStruct((E, 128), jnp.float32),
            jax.ShapeDtypeStruct((E, VW), jnp.float32),
        ],
        interpret=interpret,
    )(gs, gs, gd, gd, epack, *w)


def _scatter_stripes(s):
    return s * 624


def _make_scatter_body(width):
    def body(me_hbm, dsti_hbm, zer_hbm, out_hbm, idx_v, rows_v, accum, sem):
        c = lax.axis_index("c")
        s = lax.axis_index("s")
        wid = s * 2 + c
        base = wid * EPW
        r0 = _scatter_stripes(s)
        pltpu.sync_copy(zer_hbm.at[pl.ds(0, 624)], accum.at[pl.ds(r0, 624)])

        @pl.when(s == 15)
        def _():
            pltpu.sync_copy(zer_hbm.at[pl.ds(624, 16)],
                            accum.at[pl.ds(9984, 16)])

        plsc.subcore_barrier()
        pltpu.sync_copy(dsti_hbm.at[wid], idx_v)

        def loop(j, carry):
            pltpu.sync_copy(me_hbm.at[pl.ds(base + j * CH, CH)], rows_v)
            pltpu.sync_copy(rows_v, accum.at[idx_v.at[j]], add=True)
            return carry

        lax.fori_loop(0, NIT, loop, 0)
        plsc.subcore_barrier()
        pltpu.sync_copy(accum.at[pl.ds(r0, 624)],
                        out_hbm.at[c, pl.ds(r0, 624)])

        @pl.when(s == 15)
        def _():
            pltpu.sync_copy(accum.at[pl.ds(9984, 16)],
                            out_hbm.at[c, pl.ds(9984, 16)])

    return body


def _sc_scatter(medge, dsti, zer, width, tc_tiling):
    mesh = plsc.VectorSubcoreMesh(core_axis_name="c", subcore_axis_name="s")
    fn = pl.kernel(
        _make_scatter_body(width),
        out_type=jax.ShapeDtypeStruct((2, N, width), jnp.float32),
        mesh=mesh,
        scratch_types=[
            pltpu.VMEM((NIT, CH), jnp.int32),
            pltpu.VMEM((CH, width), jnp.float32),
            pltpu.VMEM_SHARED((N, width), jnp.float32),
            pltpu.SemaphoreType.DMA,
        ],
        compiler_params=pltpu.CompilerParams(use_tc_tiling_on_sc=tc_tiling),
    )
    return fn(medge, dsti, zer)


def _ln(x, g, b):
    mu = jnp.mean(x, axis=-1, keepdims=True)
    xc = x - mu
    var = jnp.mean(xc * xc, axis=-1, keepdims=True)
    return xc * lax.rsqrt(var + 1e-5) * g + b


def _node_post_body(ps_ref, pv_ref, s_ref, xv_ref, g0_ref, b0_ref, whf1_ref,
                    wsf1_ref, bf1_ref, wvf1_ref, whf2_ref, wsf2_ref, bf2_ref,
                    g1_ref, b1_ref, rot_ref, rob_ref, out_ref):
    accs = ps_ref[0] + ps_ref[1]
    accv = pv_ref[0] + pv_ref[1]
    cnt = jnp.maximum(accv[:, 6:7], 1.0)
    s = s_ref[...]
    sp = _ln(s + accs / cnt, g0_ref[...], b0_ref[...])
    xv = xv_ref[...]
    vp = []
    for c in range(3):
        dv = jnp.concatenate([accv[:, c:c + 1], accv[:, 3 + c:4 + c]],
                             axis=1) / cnt
        vc = jnp.concatenate([xv[:, c:c + 1], xv[:, 3 + c:4 + c]], axis=1)
        vp.append(vc + dv)
    vnch = jnp.clip(vp[0] ** 2 + vp[1] ** 2 + vp[2] ** 2, EPS, None)
    vnn = jnp.sqrt(jnp.mean(vnch, axis=-1, keepdims=True))
    vp = [v / vnn for v in vp]
    fvh = [jnp.dot(v, whf1_ref[...], preferred_element_type=jnp.float32)
           for v in vp]
    fvn = _nn(fvh[0] ** 2 + fvh[1] ** 2 + fvh[2] ** 2)
    fs = jnp.dot(jnp.concatenate([sp, fvn], axis=1), wsf1_ref[...],
                 preferred_element_type=jnp.float32) + bf1_ref[...]
    fs = jnp.maximum(fs, 0.0)
    fvo = [jnp.dot(v, wvf1_ref[...], preferred_element_type=jnp.float32)
           for v in fvh]
    gf = jax.nn.sigmoid(_nn(fvo[0] ** 2 + fvo[1] ** 2 + fvo[2] ** 2))
    fvo = [v * gf for v in fvo]
    v2 = [jnp.dot(v, whf2_ref[...], preferred_element_type=jnp.float32)
          for v in fvo]
    vn2 = _nn(v2[0] ** 2 + v2[1] ** 2 + v2[2] ** 2)
    fs2 = jnp.dot(jnp.concatenate([fs, vn2], axis=1), wsf2_ref[...],
                  preferred_element_type=jnp.float32) + bf2_ref[...]
    sf = _ln(sp + fs2, g1_ref[...], b1_ref[...])
    logit = jnp.sum(sf * rot_ref[...], axis=1, keepdims=True) + rob_ref[0, 0]
    out_ref[...] = jax.nn.sigmoid(logit)


def _node_post(ps, pv, s, xv6, w, interpret=False):
    grid = N // NBLK
    return pl.pallas_call(
        _node_post_body,
        grid=(grid,),
        in_specs=[
            pl.BlockSpec((2, NBLK, 128), lambda i: (0, i, 0)),
            pl.BlockSpec((2, NBLK, VW), lambda i: (0, i, 0)),
            pl.BlockSpec((NBLK, 128), lambda i: (i, 0)),
            pl.BlockSpec((NBLK, 6), lambda i: (i, 0)),
            _full((1, 128)),
            _full((1, 128)),
            _full((2, 4)),
            _full((132, 512)),
            _full((1, 512)),
            _full((4, 4)),
            _full((4, 4)),
            _full((516, 128)),
            _full((1, 128)),
            _full((1, 128)),
            _full((1, 128)),
            _full((1, 128)),
            _full((1, 1)),
        ],
        out_specs=pl.BlockSpec((NBLK, 1), lambda i: (i, 0)),
        out_shape=jax.ShapeDtypeStruct((N, 1), jnp.float32),
        interpret=interpret,
    )(ps, pv, s, xv6, *w)


def _edge_weights(p):
    ws1 = p["m1"]["ws_W"]
    return [
        p["m1"]["wh"][2:3, :],
        ws1[128:136, :],
        ws1[264:269, :],
        _bd(p["m1"]["wv"], 3),
        _bd(p["m2"]["wh"], 3),
        p["m2"]["ws_W"][0:128, :].astype(jnp.bfloat16),
        p["m2"]["ws_W"][128:130, :],
        p["m2"]["ws_b"].reshape(1, 128),
        _bd(p["m2"]["wv"], 3),
        _bd(p["m3"]["wh"], 3),
        p["m3"]["ws_W"][0:128, :].astype(jnp.bfloat16),
        p["m3"]["ws_W"][128:130, :],
        p["m3"]["ws_b"].reshape(1, 128),
        _bd_perm(p["m3"]["wv"]),
    ]


def _post_weights(p):
    return [
        p["ln0_g"].reshape(1, 128),
        p["ln0_b"].reshape(1, 128),
        p["ff1"]["wh"],
        p["ff1"]["ws_W"],
        p["ff1"]["ws_b"].reshape(1, 512),
        p["ff1"]["wv"],
        p["ff2"]["wh"],
        p["ff2"]["ws_W"],
        p["ff2"]["ws_b"].reshape(1, 128),
        p["ln1_g"].reshape(1, 128),
        p["ln1_b"].reshape(1, 128),
        p["ro_W"].reshape(1, 128),
        p["ro_b"].reshape(1, 1),
    ]


def kernel(x_scalar, x_vector, edge_index, edge_scalar, edge_vector, labels,
           params):
    p = params
    ws1 = p["m1"]["ws_W"]
    xv6 = x_vector.reshape(N, 6)
    epack = jnp.concatenate(
        [edge_scalar, edge_vector.reshape(E, 3),
         jnp.zeros((E, 5), jnp.float32)], axis=1)
    srci = edge_index[0].reshape(NW, NIT, CH)
    dsti = edge_index[1].reshape(NW, NIT, CH)

    s, tsrc, tdst = _node_pre(
        x_scalar, xv6, p["ls_W"], p["ls_b"].reshape(1, 128),
        ws1[0:128, :], ws1[136:264, :], p["m1"]["ws_b"].reshape(1, 128),
        p["m1"]["wh"])
    gs, gd = _sc_gather(tsrc, tdst, srci, dsti)
    meds, medv = _edge(gs, gd, epack, _edge_weights(p))
    parts = _sc_scatter(meds, dsti, jnp.zeros((640, 128), jnp.float32),
                        128, True)
    partv = _sc_scatter(medv, dsti, jnp.zeros((640, VW), jnp.float32),
                        VW, True)
    out = _node_post(parts, partv, s, xv6, _post_weights(p))
    return (out.reshape(N), labels)


# speedup vs baseline: 1.1141x; 1.1141x over previous
import jax
import jax.numpy as jnp
from jax import lax
from jax.experimental import pallas as pl
from jax.experimental.pallas import tpu as pltpu
from jax.experimental.pallas import tpu_sc as plsc

N = 10000
E = 320000
EPS = 1e-8

NW = 32
EPW = E // NW
CH = 80
NIT = EPW // CH
TW = 256
VW = 128

NBLK = 1000
EBLK = 2000


def _nn(sumsq):
    return jnp.sqrt(jnp.clip(sumsq, EPS, None))


def _bd(w, reps):
    k, n = w.shape
    m = jnp.zeros((reps * k, reps * n), w.dtype)
    for c in range(reps):
        m = m.at[c * k:(c + 1) * k, c * n:(c + 1) * n].set(w)
    return m


def _bd_perm(w):
    m = jnp.zeros((6, 6), w.dtype)
    for c in range(3):
        for i in range(2):
            for j in range(2):
                m = m.at[2 * c + i, 3 * j + c].set(w[i, j])
    return m


def _full(shape):
    return pl.BlockSpec(shape, lambda i: tuple(0 for _ in shape))


def _node_pre_body(x_ref, xv_ref, lsw_ref, lsb_ref, wa_ref, wc_ref, b1_ref,
                   wh1_ref, s_ref, tsrc_ref, tdst_ref):
    s = jnp.dot(x_ref[...], lsw_ref[...], preferred_element_type=jnp.float32)
    s = s + lsb_ref[...]
    s_ref[...] = s
    sa = jnp.dot(s, wa_ref[...], preferred_element_type=jnp.float32) + b1_ref[...]
    sc = jnp.dot(s, wc_ref[...], preferred_element_type=jnp.float32)
    xv = xv_ref[...]
    wh1 = wh1_ref[...]
    zpad = jnp.zeros((xv.shape[0], TW - 143), jnp.float32)
    pcols, qcols = [], []
    for c in range(3):
        vc = jnp.concatenate([xv[:, c:c + 1], xv[:, 3 + c:4 + c]], axis=1)
        pcols.append(jnp.dot(vc, wh1[0:2, :], preferred_element_type=jnp.float32))
        qcols.append(jnp.dot(vc, wh1[3:5, :], preferred_element_type=jnp.float32))
    tsrc_ref[...] = jnp.concatenate([sa] + pcols + [zpad], axis=1)
    tdst_ref[...] = jnp.concatenate([sc] + qcols + [zpad], axis=1)


def _node_pre(x, xv6, lsw, lsb, wa, wc, b1, wh1, interpret=False):
    grid = N // NBLK
    return pl.pallas_call(
        _node_pre_body,
        grid=(grid,),
        in_specs=[
            pl.BlockSpec((NBLK, 1433), lambda i: (i, 0)),
            pl.BlockSpec((NBLK, 6), lambda i: (i, 0)),
            _full((1433, 128)),
            _full((1, 128)),
            _full((128, 128)),
            _full((128, 128)),
            _full((1, 128)),
            _full((5, 5)),
        ],
        out_specs=[
            pl.BlockSpec((NBLK, 128), lambda i: (i, 0)),
            pl.BlockSpec((NBLK, TW), lambda i: (i, 0)),
            pl.BlockSpec((NBLK, TW), lambda i: (i, 0)),
        ],
        out_shape=[
            jax.ShapeDtypeStruct((N, 128), jnp.float32),
            jax.ShapeDtypeStruct((N, TW), jnp.float32),
            jax.ShapeDtypeStruct((N, TW), jnp.float32),
        ],
        interpret=interpret,
    )(x, xv6, lsw, lsb, wa, wc, b1, wh1)


def _sc_gather_body(tsrc_hbm, tdst_hbm, srci_hbm, dsti_hbm, gs_hbm, gd_hbm,
                    idx_v, rows_v, sem):
    c = lax.axis_index("c")
    s = lax.axis_index("s")
    wid = s * 2 + c
    base = wid * EPW

    def one_table(tab_hbm, ih_hbm, out_hbm):
        pltpu.sync_copy(ih_hbm.at[wid], idx_v)

        def body(j, carry):
            pltpu.async_copy(tab_hbm.at[idx_v.at[j]], rows_v, sem).wait()
            pltpu.sync_copy(rows_v, out_hbm.at[pl.ds(base + j * CH, CH)])
            return carry

        lax.fori_loop(0, NIT, body, 0)

    one_table(tsrc_hbm, srci_hbm, gs_hbm)
    one_table(tdst_hbm, dsti_hbm, gd_hbm)


def _sc_gather(tsrc, tdst, srci, dsti):
    mesh = plsc.VectorSubcoreMesh(core_axis_name="c", subcore_axis_name="s")
    fn = pl.kernel(
        _sc_gather_body,
        out_type=[
            jax.ShapeDtypeStruct((E, TW), jnp.float32),
            jax.ShapeDtypeStruct((E, TW), jnp.float32),
        ],
        mesh=mesh,
        scratch_types=[
            pltpu.VMEM((NIT, CH), jnp.int32),
            pltpu.VMEM((CH, TW), jnp.float32),
            pltpu.SemaphoreType.DMA,
        ],
    )
    return fn(tsrc, tdst, srci, dsti)


def _edge_body(gs_ref, gsv_ref, gd_ref, gdv_ref, ep_ref, wh1r2_ref, wb_ref,
               wd_ref, wv1_ref, wh2_ref, w2a_ref, w2b_ref, b2_ref, wv2_ref,
               wh3_ref, w3a_ref, w3b_ref, b3_ref, wv3_ref, outs_ref, outv_ref):
    gsv = gsv_ref[...]
    gdv = gdv_ref[...]
    ep = ep_ref[...]
    es = ep[:, 0:8]
    nb = es.shape[0]
    bf = jnp.bfloat16
    evt = jnp.concatenate([ep[:, 8 + c:9 + c] * wh1r2_ref[...]
                           for c in range(3)], axis=1)
    vh1 = (gsv[:, 0:15] + gdv[:, 0:15] + evt).astype(bf)
    vn1 = _nn(vh1[:, 0:5] ** 2 + vh1[:, 5:10] ** 2 + vh1[:, 10:15] ** 2)
    s1 = gs_ref[...] + gd_ref[...]
    s1 = s1 + jnp.dot(es.astype(bf), wb_ref[...],
                      preferred_element_type=jnp.float32)
    s1 = s1 + jnp.dot(vn1, wd_ref[...], preferred_element_type=jnp.float32)
    s1 = jnp.maximum(s1, 0.0)
    vo1 = jnp.dot(vh1, wv1_ref[...],
                  preferred_element_type=jnp.float32).astype(bf)
    g1 = jax.nn.sigmoid(_nn(vo1[:, 0:2] ** 2 + vo1[:, 2:4] ** 2
                            + vo1[:, 4:6] ** 2))
    vo1 = vo1 * jnp.concatenate([g1, g1, g1], axis=1)
    vh2 = jnp.dot(vo1, wh2_ref[...],
                  preferred_element_type=jnp.float32).astype(bf)
    vn2 = _nn(vh2[:, 0:2] ** 2 + vh2[:, 2:4] ** 2 + vh2[:, 4:6] ** 2)
    s2 = jnp.dot(s1.astype(bf), w2a_ref[...],
                 preferred_element_type=jnp.float32)
    s2 = s2 + jnp.dot(vn2, w2b_ref[...], preferred_element_type=jnp.float32)
    s2 = jnp.maximum(s2 + b2_ref[...], 0.0)
    vo2 = jnp.dot(vh2, wv2_ref[...],
                  preferred_element_type=jnp.float32).astype(bf)
    g2 = jax.nn.sigmoid(_nn(vo2[:, 0:2] ** 2 + vo2[:, 2:4] ** 2
                            + vo2[:, 4:6] ** 2))
    vo2 = vo2 * jnp.concatenate([g2, g2, g2], axis=1)
    vh3 = jnp.dot(vo2, wh3_ref[...],
                  preferred_element_type=jnp.float32).astype(bf)
    vn3 = _nn(vh3[:, 0:2] ** 2 + vh3[:, 2:4] ** 2 + vh3[:, 4:6] ** 2)
    s3 = jnp.dot(s2.astype(bf), w3a_ref[...],
                 preferred_element_type=jnp.float32)
    s3 = s3 + jnp.dot(vn3, w3b_ref[...], preferred_element_type=jnp.float32)
    outs_ref[...] = s3 + b3_ref[...]
    vo3 = jnp.dot(vh3, wv3_ref[...], preferred_element_type=jnp.float32)
    ones = jnp.ones((nb, 1), jnp.float32)
    zpad = jnp.zeros((nb, VW - 7), jnp.float32)
    outv_ref[...] = jnp.concatenate([vo3, ones, zpad], axis=1)


def _edge(gs, gd, epack, w, interpret=False):
    grid = E // EBLK
    return pl.pallas_call(
        _edge_body,
        grid=(grid,),
        in_specs=[
            pl.BlockSpec((EBLK, 128), lambda i: (i, 0)),
            pl.BlockSpec((EBLK, 128), lambda i: (i, 1)),
            pl.BlockSpec((EBLK, 128), lambda i: (i, 0)),
            pl.BlockSpec((EBLK, 128), lambda i: (i, 1)),
            pl.BlockSpec((EBLK, 16), lambda i: (i, 0)),
            _full((1, 5)),
            _full((8, 128)),
            _full((5, 128)),
            _full((15, 6)),
            _full((6, 6)),
            _full((128, 128)),
            _full((2, 128)),
            _full((1, 128)),
            _full((6, 6)),
            _full((6, 6)),
            _full((128, 128)),
            _full((2, 128)),
            _full((1, 128)),
            _full((6, 6)),
        ],
        out_specs=[
            pl.BlockSpec((EBLK, 128), lambda i: (i, 0)),
            pl.BlockSpec((EBLK, VW), lambda i: (i, 0)),
        ],
        out_shape=[
            jax.ShapeDtypeStruct((E, 128), jnp.float32),
            jax.ShapeDtypeStruct((E, VW), jnp.float32),
        ],
        interpret=interpret,
    )(gs, gs, gd, gd, epack, *w)


def _scatter_stripes(s):
    return s * 624


def _make_scatter_body(width):
    def body(me_hbm, dsti_hbm, zer_hbm, out_hbm, idx_v, rows_v, accum, sem):
        c = lax.axis_index("c")
        s = lax.axis_index("s")
        wid = s * 2 + c
        base = wid * EPW
        r0 = _scatter_stripes(s)
        pltpu.sync_copy(zer_hbm.at[pl.ds(0, 624)], accum.at[pl.ds(r0, 624)])

        @pl.when(s == 15)
        def _():
            pltpu.sync_copy(zer_hbm.at[pl.ds(624, 16)],
                            accum.at[pl.ds(9984, 16)])

        plsc.subcore_barrier()
        pltpu.sync_copy(dsti_hbm.at[wid], idx_v)

        def loop(j, carry):
            pltpu.sync_copy(me_hbm.at[pl.ds(base + j * CH, CH)], rows_v)
            pltpu.sync_copy(rows_v, accum.at[idx_v.at[j]], add=True)
            return carry

        lax.fori_loop(0, NIT, loop, 0)
        plsc.subcore_barrier()
        pltpu.sync_copy(accum.at[pl.ds(r0, 624)],
                        out_hbm.at[c, pl.ds(r0, 624)])

        @pl.when(s == 15)
        def _():
            pltpu.sync_copy(accum.at[pl.ds(9984, 16)],
                            out_hbm.at[c, pl.ds(9984, 16)])

    return body


def _sc_scatter(medge, dsti, zer, width, tc_tiling):
    mesh = plsc.VectorSubcoreMesh(core_axis_name="c", subcore_axis_name="s")
    fn = pl.kernel(
        _make_scatter_body(width),
        out_type=jax.ShapeDtypeStruct((2, N, width), jnp.float32),
        mesh=mesh,
        scratch_types=[
            pltpu.VMEM((NIT, CH), jnp.int32),
            pltpu.VMEM((CH, width), jnp.float32),
            pltpu.VMEM_SHARED((N, width), jnp.float32),
            pltpu.SemaphoreType.DMA,
        ],
        compiler_params=pltpu.CompilerParams(use_tc_tiling_on_sc=tc_tiling),
    )
    return fn(medge, dsti, zer)


def _ln(x, g, b):
    mu = jnp.mean(x, axis=-1, keepdims=True)
    xc = x - mu
    var = jnp.mean(xc * xc, axis=-1, keepdims=True)
    return xc * lax.rsqrt(var + 1e-5) * g + b


def _node_post_body(ps_ref, pv_ref, s_ref, xv_ref, g0_ref, b0_ref, whf1_ref,
                    wsf1_ref, bf1_ref, wvf1_ref, whf2_ref, wsf2_ref, bf2_ref,
                    g1_ref, b1_ref, rot_ref, rob_ref, out_ref):
    accs = ps_ref[0] + ps_ref[1]
    accv = pv_ref[0] + pv_ref[1]
    cnt = jnp.maximum(accv[:, 6:7], 1.0)
    s = s_ref[...]
    sp = _ln(s + accs / cnt, g0_ref[...], b0_ref[...])
    xv = xv_ref[...]
    vp = []
    for c in range(3):
        dv = jnp.concatenate([accv[:, c:c + 1], accv[:, 3 + c:4 + c]],
                             axis=1) / cnt
        vc = jnp.concatenate([xv[:, c:c + 1], xv[:, 3 + c:4 + c]], axis=1)
        vp.append(vc + dv)
    vnch = jnp.clip(vp[0] ** 2 + vp[1] ** 2 + vp[2] ** 2, EPS, None)
    vnn = jnp.sqrt(jnp.mean(vnch, axis=-1, keepdims=True))
    vp = [v / vnn for v in vp]
    fvh = [jnp.dot(v, whf1_ref[...], preferred_element_type=jnp.float32)
           for v in vp]
    fvn = _nn(fvh[0] ** 2 + fvh[1] ** 2 + fvh[2] ** 2)
    fs = jnp.dot(jnp.concatenate([sp, fvn], axis=1), wsf1_ref[...],
                 preferred_element_type=jnp.float32) + bf1_ref[...]
    fs = jnp.maximum(fs, 0.0)
    fvo = [jnp.dot(v, wvf1_ref[...], preferred_element_type=jnp.float32)
           for v in fvh]
    gf = jax.nn.sigmoid(_nn(fvo[0] ** 2 + fvo[1] ** 2 + fvo[2] ** 2))
    fvo = [v * gf for v in fvo]
    v2 = [jnp.dot(v, whf2_ref[...], preferred_element_type=jnp.float32)
          for v in fvo]
    vn2 = _nn(v2[0] ** 2 + v2[1] ** 2 + v2[2] ** 2)
    fs2 = jnp.dot(jnp.concatenate([fs, vn2], axis=1), wsf2_ref[...],
                  preferred_element_type=jnp.float32) + bf2_ref[...]
    sf = _ln(sp + fs2, g1_ref[...], b1_ref[...])
    logit = jnp.sum(sf * rot_ref[...], axis=1, keepdims=True) + rob_ref[0, 0]
    out_ref[...] = jax.nn.sigmoid(logit)


def _node_post(ps, pv, s, xv6, w, interpret=False):
    grid = N // NBLK
    return pl.pallas_call(
        _node_post_body,
        grid=(grid,),
        in_specs=[
            pl.BlockSpec((2, NBLK, 128), lambda i: (0, i, 0)),
            pl.BlockSpec((2, NBLK, VW), lambda i: (0, i, 0)),
            pl.BlockSpec((NBLK, 128), lambda i: (i, 0)),
            pl.BlockSpec((NBLK, 6), lambda i: (i, 0)),
            _full((1, 128)),
            _full((1, 128)),
            _full((2, 4)),
            _full((132, 512)),
            _full((1, 512)),
            _full((4, 4)),
            _full((4, 4)),
            _full((516, 128)),
            _full((1, 128)),
            _full((1, 128)),
            _full((1, 128)),
            _full((1, 128)),
            _full((1, 1)),
        ],
        out_specs=pl.BlockSpec((NBLK, 1), lambda i: (i, 0)),
        out_shape=jax.ShapeDtypeStruct((N, 1), jnp.float32),
        interpret=interpret,
    )(ps, pv, s, xv6, *w)


def _edge_weights(p):
    ws1 = p["m1"]["ws_W"]
    return [
        p["m1"]["wh"][2:3, :],
        ws1[128:136, :].astype(jnp.bfloat16),
        ws1[264:269, :].astype(jnp.bfloat16),
        _bd(p["m1"]["wv"], 3).astype(jnp.bfloat16),
        _bd(p["m2"]["wh"], 3).astype(jnp.bfloat16),
        p["m2"]["ws_W"][0:128, :].astype(jnp.bfloat16),
        p["m2"]["ws_W"][128:130, :].astype(jnp.bfloat16),
        p["m2"]["ws_b"].reshape(1, 128),
        _bd(p["m2"]["wv"], 3).astype(jnp.bfloat16),
        _bd(p["m3"]["wh"], 3).astype(jnp.bfloat16),
        p["m3"]["ws_W"][0:128, :].astype(jnp.bfloat16),
        p["m3"]["ws_W"][128:130, :].astype(jnp.bfloat16),
        p["m3"]["ws_b"].reshape(1, 128),
        _bd_perm(p["m3"]["wv"]).astype(jnp.bfloat16),
    ]


def _post_weights(p):
    return [
        p["ln0_g"].reshape(1, 128),
        p["ln0_b"].reshape(1, 128),
        p["ff1"]["wh"],
        p["ff1"]["ws_W"],
        p["ff1"]["ws_b"].reshape(1, 512),
        p["ff1"]["wv"],
        p["ff2"]["wh"],
        p["ff2"]["ws_W"],
        p["ff2"]["ws_b"].reshape(1, 128),
        p["ln1_g"].reshape(1, 128),
        p["ln1_b"].reshape(1, 128),
        p["ro_W"].reshape(1, 128),
        p["ro_b"].reshape(1, 1),
    ]


def kernel(x_scalar, x_vector, edge_index, edge_scalar, edge_vector, labels,
           params):
    p = params
    ws1 = p["m1"]["ws_W"]
    xv6 = x_vector.reshape(N, 6)
    epack = jnp.concatenate(
        [edge_scalar, edge_vector.reshape(E, 3),
         jnp.zeros((E, 5), jnp.float32)], axis=1)
    srci = edge_index[0].reshape(NW, NIT, CH)
    dsti = edge_index[1].reshape(NW, NIT, CH)

    s, tsrc, tdst = _node_pre(
        x_scalar, xv6, p["ls_W"], p["ls_b"].reshape(1, 128),
        ws1[0:128, :], ws1[136:264, :], p["m1"]["ws_b"].reshape(1, 128),
        p["m1"]["wh"])
    gs, gd = _sc_gather(tsrc, tdst, srci, dsti)
    meds, medv = _edge(gs, gd, epack, _edge_weights(p))
    parts = _sc_scatter(meds, dsti, jnp.zeros((640, 128), jnp.float32),
                        128, True)
    partv = _sc_scatter(medv, dsti, jnp.zeros((640, VW), jnp.float32),
                        VW, True)
    out = _node_post(parts, partv, s, xv6, _post_weights(p))
    return (out.reshape(N), labels)


# speedup vs baseline: 1.1654x; 1.0460x over previous
import jax
import jax.numpy as jnp
from jax import lax
from jax.experimental import pallas as pl
from jax.experimental.pallas import tpu as pltpu
from jax.experimental.pallas import tpu_sc as plsc

N = 10000
E = 320000
EPS = 1e-8

NW = 32
EPW = E // NW
CH = 80
NIT = EPW // CH
TW = 256
VW = 128

NBLK = 1000
EBLK = 2000


def _nn(sumsq):
    return jnp.sqrt(jnp.clip(sumsq, EPS, None))


def _bd(w, reps):
    k, n = w.shape
    m = jnp.zeros((reps * k, reps * n), w.dtype)
    for c in range(reps):
        m = m.at[c * k:(c + 1) * k, c * n:(c + 1) * n].set(w)
    return m


def _bd_perm(w):
    m = jnp.zeros((6, 6), w.dtype)
    for c in range(3):
        for i in range(2):
            for j in range(2):
                m = m.at[2 * c + i, 3 * j + c].set(w[i, j])
    return m


def _full(shape):
    return pl.BlockSpec(shape, lambda i: tuple(0 for _ in shape))


def _node_pre_body(x_ref, xv_ref, lsw_ref, lsb_ref, wa_ref, wc_ref, b1_ref,
                   wh1_ref, s_ref, tsrc_ref, tdst_ref):
    s = jnp.dot(x_ref[...], lsw_ref[...], preferred_element_type=jnp.float32)
    s = s + lsb_ref[...]
    s_ref[...] = s
    sa = jnp.dot(s, wa_ref[...], preferred_element_type=jnp.float32) + b1_ref[...]
    sc = jnp.dot(s, wc_ref[...], preferred_element_type=jnp.float32)
    xv = xv_ref[...]
    wh1 = wh1_ref[...]
    zpad = jnp.zeros((xv.shape[0], TW - 143), jnp.float32)
    pcols, qcols = [], []
    for c in range(3):
        vc = jnp.concatenate([xv[:, c:c + 1], xv[:, 3 + c:4 + c]], axis=1)
        pcols.append(jnp.dot(vc, wh1[0:2, :], preferred_element_type=jnp.float32))
        qcols.append(jnp.dot(vc, wh1[3:5, :], preferred_element_type=jnp.float32))
    tsrc_ref[...] = jnp.concatenate([sa] + pcols + [zpad], axis=1)
    tdst_ref[...] = jnp.concatenate([sc] + qcols + [zpad], axis=1)


def _node_pre(x, xv6, lsw, lsb, wa, wc, b1, wh1, interpret=False):
    grid = N // NBLK
    return pl.pallas_call(
        _node_pre_body,
        grid=(grid,),
        in_specs=[
            pl.BlockSpec((NBLK, 1433), lambda i: (i, 0)),
            pl.BlockSpec((NBLK, 6), lambda i: (i, 0)),
            _full((1433, 128)),
            _full((1, 128)),
            _full((128, 128)),
            _full((128, 128)),
            _full((1, 128)),
            _full((5, 5)),
        ],
        out_specs=[
            pl.BlockSpec((NBLK, 128), lambda i: (i, 0)),
            pl.BlockSpec((NBLK, TW), lambda i: (i, 0)),
            pl.BlockSpec((NBLK, TW), lambda i: (i, 0)),
        ],
        out_shape=[
            jax.ShapeDtypeStruct((N, 128), jnp.float32),
            jax.ShapeDtypeStruct((N, TW), jnp.float32),
            jax.ShapeDtypeStruct((N, TW), jnp.float32),
        ],
        interpret=interpret,
    )(x, xv6, lsw, lsb, wa, wc, b1, wh1)


def _sc_gather_body(tsrc_hbm, tdst_hbm, srci_hbm, dsti_hbm, gs_hbm, gd_hbm,
                    idx_v, rows0, rows1, sem0, sem1):
    c = lax.axis_index("c")
    s = lax.axis_index("s")
    wid = s * 2 + c
    base = wid * EPW

    def one_table(tab_hbm, ih_hbm, out_hbm):
        pltpu.sync_copy(ih_hbm.at[wid], idx_v)
        pltpu.async_copy(tab_hbm.at[idx_v.at[0]], rows0, sem0)

        def body(k, carry):
            j = 2 * k
            pltpu.make_async_copy(tab_hbm.at[idx_v.at[j]], rows0, sem0).wait()
            pltpu.async_copy(tab_hbm.at[idx_v.at[j + 1]], rows1, sem1)
            pltpu.sync_copy(rows0, out_hbm.at[pl.ds(base + j * CH, CH)])
            pltpu.make_async_copy(tab_hbm.at[idx_v.at[j + 1]], rows1,
                                  sem1).wait()
            pltpu.async_copy(tab_hbm.at[idx_v.at[j + 2]], rows0, sem0)
            pltpu.sync_copy(rows1, out_hbm.at[pl.ds(base + (j + 1) * CH, CH)])
            return carry

        lax.fori_loop(0, (NIT - 1) // 2, body, 0)
        pltpu.make_async_copy(tab_hbm.at[idx_v.at[NIT - 1]], rows0,
                              sem0).wait()
        pltpu.sync_copy(rows0, out_hbm.at[pl.ds(base + (NIT - 1) * CH, CH)])

    one_table(tsrc_hbm, srci_hbm, gs_hbm)
    one_table(tdst_hbm, dsti_hbm, gd_hbm)


def _sc_gather(tsrc, tdst, srci, dsti):
    mesh = plsc.VectorSubcoreMesh(core_axis_name="c", subcore_axis_name="s")
    fn = pl.kernel(
        _sc_gather_body,
        out_type=[
            jax.ShapeDtypeStruct((E, TW), jnp.float32),
            jax.ShapeDtypeStruct((E, TW), jnp.float32),
        ],
        mesh=mesh,
        scratch_types=[
            pltpu.VMEM((NIT, CH), jnp.int32),
            pltpu.VMEM((CH, TW), jnp.float32),
            pltpu.VMEM((CH, TW), jnp.float32),
            pltpu.SemaphoreType.DMA,
            pltpu.SemaphoreType.DMA,
        ],
    )
    return fn(tsrc, tdst, srci, dsti)


def _edge_body(gs_ref, gsv_ref, gd_ref, gdv_ref, ep_ref, wh1r2_ref, wb_ref,
               wd_ref, wv1_ref, wh2_ref, w2a_ref, w2b_ref, b2_ref, wv2_ref,
               wh3_ref, w3a_ref, w3b_ref, b3_ref, wv3_ref, outs_ref, outv_ref):
    gsv = gsv_ref[...]
    gdv = gdv_ref[...]
    ep = ep_ref[...]
    es = ep[:, 0:8]
    nb = es.shape[0]
    bf = jnp.bfloat16
    evt = jnp.concatenate([ep[:, 8 + c:9 + c] * wh1r2_ref[...]
                           for c in range(3)], axis=1)
    vh1 = (gsv[:, 0:15] + gdv[:, 0:15] + evt).astype(bf)
    vn1 = _nn(vh1[:, 0:5] ** 2 + vh1[:, 5:10] ** 2 + vh1[:, 10:15] ** 2)
    s1 = gs_ref[...] + gd_ref[...]
    s1 = s1 + jnp.dot(es.astype(bf), wb_ref[...],
                      preferred_element_type=jnp.float32)
    s1 = s1 + jnp.dot(vn1, wd_ref[...], preferred_element_type=jnp.float32)
    s1 = jnp.maximum(s1, 0.0)
    vo1 = jnp.dot(vh1, wv1_ref[...],
                  preferred_element_type=jnp.float32).astype(bf)
    g1 = jax.nn.sigmoid(_nn(vo1[:, 0:2] ** 2 + vo1[:, 2:4] ** 2
                            + vo1[:, 4:6] ** 2))
    vo1 = vo1 * jnp.concatenate([g1, g1, g1], axis=1)
    vh2 = jnp.dot(vo1, wh2_ref[...],
                  preferred_element_type=jnp.float32).astype(bf)
    vn2 = _nn(vh2[:, 0:2] ** 2 + vh2[:, 2:4] ** 2 + vh2[:, 4:6] ** 2)
    s2 = jnp.dot(s1.astype(bf), w2a_ref[...],
                 preferred_element_type=jnp.float32)
    s2 = s2 + jnp.dot(vn2, w2b_ref[...], preferred_element_type=jnp.float32)
    s2 = jnp.maximum(s2 + b2_ref[...], 0.0)
    vo2 = jnp.dot(vh2, wv2_ref[...],
                  preferred_element_type=jnp.float32).astype(bf)
    g2 = jax.nn.sigmoid(_nn(vo2[:, 0:2] ** 2 + vo2[:, 2:4] ** 2
                            + vo2[:, 4:6] ** 2))
    vo2 = vo2 * jnp.concatenate([g2, g2, g2], axis=1)
    vh3 = jnp.dot(vo2, wh3_ref[...],
                  preferred_element_type=jnp.float32).astype(bf)
    vn3 = _nn(vh3[:, 0:2] ** 2 + vh3[:, 2:4] ** 2 + vh3[:, 4:6] ** 2)
    s3 = jnp.dot(s2.astype(bf), w3a_ref[...],
                 preferred_element_type=jnp.float32)
    s3 = s3 + jnp.dot(vn3, w3b_ref[...], preferred_element_type=jnp.float32)
    outs_ref[...] = s3 + b3_ref[...]
    vo3 = jnp.dot(vh3, wv3_ref[...], preferred_element_type=jnp.float32)
    ones = jnp.ones((nb, 1), jnp.float32)
    zpad = jnp.zeros((nb, VW - 7), jnp.float32)
    outv_ref[...] = jnp.concatenate([vo3, ones, zpad], axis=1)


def _edge(gs, gd, epack, w, interpret=False):
    grid = E // EBLK
    return pl.pallas_call(
        _edge_body,
        grid=(grid,),
        in_specs=[
            pl.BlockSpec((EBLK, 128), lambda i: (i, 0)),
            pl.BlockSpec((EBLK, 128), lambda i: (i, 1)),
            pl.BlockSpec((EBLK, 128), lambda i: (i, 0)),
            pl.BlockSpec((EBLK, 128), lambda i: (i, 1)),
            pl.BlockSpec((EBLK, 16), lambda i: (i, 0)),
            _full((1, 5)),
            _full((8, 128)),
            _full((5, 128)),
            _full((15, 6)),
            _full((6, 6)),
            _full((128, 128)),
            _full((2, 128)),
            _full((1, 128)),
            _full((6, 6)),
            _full((6, 6)),
            _full((128, 128)),
            _full((2, 128)),
            _full((1, 128)),
            _full((6, 6)),
        ],
        out_specs=[
            pl.BlockSpec((EBLK, 128), lambda i: (i, 0)),
            pl.BlockSpec((EBLK, VW), lambda i: (i, 0)),
        ],
        out_shape=[
            jax.ShapeDtypeStruct((E, 128), jnp.float32),
            jax.ShapeDtypeStruct((E, VW), jnp.float32),
        ],
        interpret=interpret,
    )(gs, gs, gd, gd, epack, *w)


def _scatter_stripes(s):
    return s * 624


def _make_scatter_body(width):
    def body(me_hbm, dsti_hbm, zer_hbm, out_hbm, idx_v, rows_v, accum, sem):
        c = lax.axis_index("c")
        s = lax.axis_index("s")
        wid = s * 2 + c
        base = wid * EPW
        r0 = _scatter_stripes(s)
        pltpu.sync_copy(zer_hbm.at[pl.ds(0, 624)], accum.at[pl.ds(r0, 624)])

        @pl.when(s == 15)
        def _():
            pltpu.sync_copy(zer_hbm.at[pl.ds(624, 16)],
                            accum.at[pl.ds(9984, 16)])

        plsc.subcore_barrier()
        pltpu.sync_copy(dsti_hbm.at[wid], idx_v)

        def loop(j, carry):
            pltpu.sync_copy(me_hbm.at[pl.ds(base + j * CH, CH)], rows_v)
            pltpu.sync_copy(rows_v, accum.at[idx_v.at[j]], add=True)
            return carry

        lax.fori_loop(0, NIT, loop, 0)
        plsc.subcore_barrier()
        pltpu.sync_copy(accum.at[pl.ds(r0, 624)],
                        out_hbm.at[c, pl.ds(r0, 624)])

        @pl.when(s == 15)
        def _():
            pltpu.sync_copy(accum.at[pl.ds(9984, 16)],
                            out_hbm.at[c, pl.ds(9984, 16)])

    return body


def _sc_scatter(medge, dsti, zer, width, tc_tiling):
    mesh = plsc.VectorSubcoreMesh(core_axis_name="c", subcore_axis_name="s")
    fn = pl.kernel(
        _make_scatter_body(width),
        out_type=jax.ShapeDtypeStruct((2, N, width), jnp.float32),
        mesh=mesh,
        scratch_types=[
            pltpu.VMEM((NIT, CH), jnp.int32),
            pltpu.VMEM((CH, width), jnp.float32),
            pltpu.VMEM_SHARED((N, width), jnp.float32),
            pltpu.SemaphoreType.DMA,
        ],
        compiler_params=pltpu.CompilerParams(use_tc_tiling_on_sc=tc_tiling),
    )
    return fn(medge, dsti, zer)


def _ln(x, g, b):
    mu = jnp.mean(x, axis=-1, keepdims=True)
    xc = x - mu
    var = jnp.mean(xc * xc, axis=-1, keepdims=True)
    return xc * lax.rsqrt(var + 1e-5) * g + b


def _node_post_body(ps_ref, pv_ref, s_ref, xv_ref, g0_ref, b0_ref, whf1_ref,
                    wsf1_ref, bf1_ref, wvf1_ref, whf2_ref, wsf2_ref, bf2_ref,
                    g1_ref, b1_ref, rot_ref, rob_ref, out_ref):
    accs = ps_ref[0] + ps_ref[1]
    accv = pv_ref[0] + pv_ref[1]
    cnt = jnp.maximum(accv[:, 6:7], 1.0)
    s = s_ref[...]
    sp = _ln(s + accs / cnt, g0_ref[...], b0_ref[...])
    xv = xv_ref[...]
    vp = []
    for c in range(3):
        dv = jnp.concatenate([accv[:, c:c + 1], accv[:, 3 + c:4 + c]],
                             axis=1) / cnt
        vc = jnp.concatenate([xv[:, c:c + 1], xv[:, 3 + c:4 + c]], axis=1)
        vp.append(vc + dv)
    vnch = jnp.clip(vp[0] ** 2 + vp[1] ** 2 + vp[2] ** 2, EPS, None)
    vnn = jnp.sqrt(jnp.mean(vnch, axis=-1, keepdims=True))
    vp = [v / vnn for v in vp]
    fvh = [jnp.dot(v, whf1_ref[...], preferred_element_type=jnp.float32)
           for v in vp]
    fvn = _nn(fvh[0] ** 2 + fvh[1] ** 2 + fvh[2] ** 2)
    fs = jnp.dot(jnp.concatenate([sp, fvn], axis=1), wsf1_ref[...],
                 preferred_element_type=jnp.float32) + bf1_ref[...]
    fs = jnp.maximum(fs, 0.0)
    fvo = [jnp.dot(v, wvf1_ref[...], preferred_element_type=jnp.float32)
           for v in fvh]
    gf = jax.nn.sigmoid(_nn(fvo[0] ** 2 + fvo[1] ** 2 + fvo[2] ** 2))
    fvo = [v * gf for v in fvo]
    v2 = [jnp.dot(v, whf2_ref[...], preferred_element_type=jnp.float32)
          for v in fvo]
    vn2 = _nn(v2[0] ** 2 + v2[1] ** 2 + v2[2] ** 2)
    fs2 = jnp.dot(jnp.concatenate([fs, vn2], axis=1), wsf2_ref[...],
                  preferred_element_type=jnp.float32) + bf2_ref[...]
    sf = _ln(sp + fs2, g1_ref[...], b1_ref[...])
    logit = jnp.sum(sf * rot_ref[...], axis=1, keepdims=True) + rob_ref[0, 0]
    out_ref[...] = jax.nn.sigmoid(logit)


def _node_post(ps, pv, s, xv6, w, interpret=False):
    grid = N // NBLK
    return pl.pallas_call(
        _node_post_body,
        grid=(grid,),
        in_specs=[
            pl.BlockSpec((2, NBLK, 128), lambda i: (0, i, 0)),
            pl.BlockSpec((2, NBLK, VW), lambda i: (0, i, 0)),
            pl.BlockSpec((NBLK, 128), lambda i: (i, 0)),
            pl.BlockSpec((NBLK, 6), lambda i: (i, 0)),
            _full((1, 128)),
            _full((1, 128)),
            _full((2, 4)),
            _full((132, 512)),
            _full((1, 512)),
            _full((4, 4)),
            _full((4, 4)),
            _full((516, 128)),
            _full((1, 128)),
            _full((1, 128)),
            _full((1, 128)),
            _full((1, 128)),
            _full((1, 1)),
        ],
        out_specs=pl.BlockSpec((NBLK, 1), lambda i: (i, 0)),
        out_shape=jax.ShapeDtypeStruct((N, 1), jnp.float32),
        interpret=interpret,
    )(ps, pv, s, xv6, *w)


def _edge_weights(p):
    ws1 = p["m1"]["ws_W"]
    return [
        p["m1"]["wh"][2:3, :],
        ws1[128:136, :].astype(jnp.bfloat16),
        ws1[264:269, :].astype(jnp.bfloat16),
        _bd(p["m1"]["wv"], 3).astype(jnp.bfloat16),
        _bd(p["m2"]["wh"], 3).astype(jnp.bfloat16),
        p["m2"]["ws_W"][0:128, :].astype(jnp.bfloat16),
        p["m2"]["ws_W"][128:130, :].astype(jnp.bfloat16),
        p["m2"]["ws_b"].reshape(1, 128),
        _bd(p["m2"]["wv"], 3).astype(jnp.bfloat16),
        _bd(p["m3"]["wh"], 3).astype(jnp.bfloat16),
        p["m3"]["ws_W"][0:128, :].astype(jnp.bfloat16),
        p["m3"]["ws_W"][128:130, :].astype(jnp.bfloat16),
        p["m3"]["ws_b"].reshape(1, 128),
        _bd_perm(p["m3"]["wv"]).astype(jnp.bfloat16),
    ]


def _post_weights(p):
    return [
        p["ln0_g"].reshape(1, 128),
        p["ln0_b"].reshape(1, 128),
        p["ff1"]["wh"],
        p["ff1"]["ws_W"],
        p["ff1"]["ws_b"].reshape(1, 512),
        p["ff1"]["wv"],
        p["ff2"]["wh"],
        p["ff2"]["ws_W"],
        p["ff2"]["ws_b"].reshape(1, 128),
        p["ln1_g"].reshape(1, 128),
        p["ln1_b"].reshape(1, 128),
        p["ro_W"].reshape(1, 128),
        p["ro_b"].reshape(1, 1),
    ]


def kernel(x_scalar, x_vector, edge_index, edge_scalar, edge_vector, labels,
           params):
    p = params
    ws1 = p["m1"]["ws_W"]
    xv6 = x_vector.reshape(N, 6)
    epack = jnp.concatenate(
        [edge_scalar, edge_vector.reshape(E, 3),
         jnp.zeros((E, 5), jnp.float32)], axis=1)
    srci = edge_index[0].reshape(NW, NIT, CH)
    dsti = edge_index[1].reshape(NW, NIT, CH)

    s, tsrc, tdst = _node_pre(
        x_scalar, xv6, p["ls_W"], p["ls_b"].reshape(1, 128),
        ws1[0:128, :], ws1[136:264, :], p["m1"]["ws_b"].reshape(1, 128),
        p["m1"]["wh"])
    gs, gd = _sc_gather(tsrc, tdst, srci, dsti)
    meds, medv = _edge(gs, gd, epack, _edge_weights(p))
    parts = _sc_scatter(meds, dsti, jnp.zeros((640, 128), jnp.float32),
                        128, True)
    partv = _sc_scatter(medv, dsti, jnp.zeros((640, VW), jnp.float32),
                        VW, True)
    out = _node_post(parts, partv, s, xv6, _post_weights(p))
    return (out.reshape(N), labels)
